# baseline (speedup 1.0000x reference)
import functools

import jax
import jax.numpy as jnp
from jax import lax
from jax.experimental import pallas as pl
from jax.experimental.pallas import tpu as pltpu
from jax.experimental.pallas import tpu_sc as plsc

HEADS = 8
DIM_HEAD = 64
DIM = 384
INNER = HEADS * DIM_HEAD
TOPK = 16
HW = 32
P = HW * HW
NSEL = TOPK * TOPK


def _dot(a, b, dims):
    return lax.dot_general(a, b, (dims, ((), ())),
                           preferred_element_type=jnp.float32)


def _stage1_body(ctx_ref, qs_ref, g_ctx_ref, b_ctx_ref, g_qs_ref, b_qs_ref,
                 wq_ref, wkv_ref, qn_ref, kvt_ref, qsn_ref, idx_ref):
    ctx = ctx_ref[0].reshape(DIM, P)
    qs = qs_ref[0].reshape(DIM, P)

    def chan_ln(x, g, b):
        mean = jnp.mean(x, axis=0, keepdims=True)
        var = jnp.mean((x - mean) ** 2, axis=0, keepdims=True)
        return (x - mean) * lax.rsqrt(var + 1e-5) * g + b

    ctxn = chan_ln(ctx, g_ctx_ref[...], b_ctx_ref[...])
    qsn = chan_ln(qs, g_qs_ref[...], b_qs_ref[...])
    qsn_ref[0] = qsn

    kv = _dot(wkv_ref[...], ctxn, ((1,), (0,)))
    q = _dot(wq_ref[...], qsn, ((1,), (0,)))
    k = kv[:INNER]
    v = kv[INNER:]


    i0 = jax.lax.broadcasted_iota(jnp.int32, (P, HW), 0)
    i1 = jax.lax.broadcasted_iota(jnp.int32, (P, HW), 1)
    Rm = (i0 // HW == i1).astype(jnp.float32)
    Cm = (i0 % HW == i1).astype(jnp.float32)

    hh = jax.lax.broadcasted_iota(jnp.int32, (HW, HW), 0)
    ww = jax.lax.broadcasted_iota(jnp.int32, (HW, HW), 1)
    LT = (ww < hh).astype(jnp.float32)
    Id64 = (jax.lax.broadcasted_iota(jnp.int32, (DIM_HEAD, DIM_HEAD), 0) ==
            jax.lax.broadcasted_iota(jnp.int32, (DIM_HEAD, DIM_HEAD), 1)
            ).astype(jnp.float32)
    Id16 = (jax.lax.broadcasted_iota(jnp.int32, (TOPK, TOPK), 0) ==
            jax.lax.broadcasted_iota(jnp.int32, (TOPK, TOPK), 1)
            ).astype(jnp.float32)
    ii_col = jax.lax.broadcasted_iota(jnp.int32, (HW, 1), 0).astype(jnp.float32)
    ss_row = jax.lax.broadcasted_iota(jnp.int32, (HW, TOPK), 1).astype(jnp.float32)

    pid = pl.program_id(0)
    qn_all = []
    idx_all = []
    for h in range(HEADS):
        qh = q[h * DIM_HEAD:(h + 1) * DIM_HEAD]
        kh = k[h * DIM_HEAD:(h + 1) * DIM_HEAD]
        vh = v[h * DIM_HEAD:(h + 1) * DIM_HEAD]
        qh = qh * lax.rsqrt(jnp.maximum(
            jnp.sum(qh * qh, axis=0, keepdims=True), 1e-24))
        kh = kh * lax.rsqrt(jnp.maximum(
            jnp.sum(kh * kh, axis=0, keepdims=True), 1e-24))
        qn_all.append(qh)

        kt = jnp.transpose(kh)
        vt = jnp.transpose(vh)
        kvt_ref[0, h * P:(h + 1) * P, :] = jnp.concatenate([kt, vt], axis=1)

        q_probe = jnp.sum(jnp.abs(qh), axis=1, keepdims=True)
        s_pos = jnp.sum(jnp.abs(kh) * q_probe, axis=0, keepdims=True)

        def topk_idx(sel_mat):
            s_row = _dot(s_pos, sel_mat, ((1,), (0,)))
            s_col = _dot(sel_mat, s_pos, ((0,), (1,)))
            vi = jnp.broadcast_to(s_col, (HW, HW))
            vj = jnp.broadcast_to(s_row, (HW, HW))
            beats = (vj > vi) | ((vj == vi) & (ww < hh))
            rank = jnp.sum(beats.astype(jnp.float32), axis=1, keepdims=True)
            sel = (rank < TOPK).astype(jnp.float32)
            pos = _dot(LT, sel, ((1,), (0,)))
            G = sel * (jnp.broadcast_to(pos, (HW, TOPK)) == ss_row)
            return _dot(G, ii_col, ((0,), (0,)))

        ridx = topk_idx(Rm)
        cidx = topk_idx(Cm)
        cidx_row = _dot(cidx, Id16, ((0,), (0,)))
        idx_f = ridx * float(HW) + jnp.broadcast_to(cidx_row, (TOPK, TOPK))
        base = (pid * HEADS + h) * P
        idx_all.append(idx_f.astype(jnp.int32) + base)

    qn_ref[0] = jnp.concatenate(qn_all, axis=0)
    idx_ref[0] = jnp.stack(idx_all, axis=0)


def _sc_gather(kvflat, idx):
    mesh = plsc.VectorSubcoreMesh(core_axis_name="c", subcore_axis_name="s")
    info = plsc.get_sparse_core_info()
    nc = info.num_cores

    @functools.partial(
        pl.kernel, mesh=mesh,
        out_type=jax.ShapeDtypeStruct((32, NSEL, 2 * DIM_HEAD), jnp.float32),
        scratch_types=[
            pltpu.VMEM((2, 128), jnp.int32),
            pltpu.VMEM((NSEL, 2 * DIM_HEAD), jnp.float32),
            pltpu.SemaphoreType.DMA,
        ],
    )
    def gather(kv_hbm, idx_hbm, kvg_hbm, idx_v, buf, sem):
        wid = lax.axis_index("s") * nc + lax.axis_index("c")
        pltpu.sync_copy(idx_hbm.at[wid], idx_v)
        copies = []
        for half in range(2):
            dst = pl.ds(half * 128, 128)
            copies.append(pltpu.async_copy(
                kv_hbm.at[idx_v.at[half]], buf.at[dst], sem))
        for c in copies:
            c.wait()
        pltpu.sync_copy(buf, kvg_hbm.at[wid])

    return gather(kvflat, idx)


def _stage2_body(qn_ref, kvg_ref, wout_ref, qsn_ref, gamma_ref, out_ref):
    ones_ns = jnp.ones((1, NSEL), jnp.float32)
    inner_parts = []
    for h in range(HEADS):
        qh = qn_ref[0][h * DIM_HEAD:(h + 1) * DIM_HEAD]
        kvg = kvg_ref[0, h]
        kg = kvg[:, :DIM_HEAD]
        vg = kvg[:, DIM_HEAD:]
        sim = _dot(qh, kg, ((0,), (1,)))
        e = jnp.exp(sim)
        s_row = _dot(ones_ns, e, ((1,), (1,)))
        o = _dot(vg, e, ((0,), (1,)))
        inner_parts.append(o * (1.0 / s_row))
    inner = jnp.concatenate(inner_parts, axis=0)
    proj = _dot(wout_ref[...], inner, ((1,), (0,)))
    out_ref[0] = gamma_ref[0, 0] * proj + qsn_ref[0]


def kernel(context, query_source, g_ctx, b_ctx, g_qs, b_qs, W_q, W_kv, W_out,
           gamma):
    b = context.shape[0]
    B = b * HEADS
    g_ctx = g_ctx.reshape(DIM, 1)
    b_ctx = b_ctx.reshape(DIM, 1)
    g_qs = g_qs.reshape(DIM, 1)
    b_qs = b_qs.reshape(DIM, 1)

    full = lambda shape: pl.BlockSpec(shape, lambda i: (0,) * len(shape))
    batch3 = lambda shape: pl.BlockSpec(shape, lambda i: (i, 0, 0))
    batch4 = lambda shape: pl.BlockSpec(shape, lambda i: (i, 0, 0, 0))

    qn, kvt, qsn, idx = pl.pallas_call(
        _stage1_body,
        grid=(b,),
        in_specs=[
            batch4((1, DIM, HW, HW)), batch4((1, DIM, HW, HW)),
            full((DIM, 1)), full((DIM, 1)), full((DIM, 1)), full((DIM, 1)),
            full((INNER, DIM)), full((2 * INNER, DIM)),
        ],
        out_specs=[
            batch3((1, INNER, P)),
            batch3((1, HEADS * P, 2 * DIM_HEAD)),
            batch3((1, DIM, P)),
            batch4((1, HEADS, TOPK, TOPK)),
        ],
        out_shape=[
            jax.ShapeDtypeStruct((b, INNER, P), jnp.float32),
            jax.ShapeDtypeStruct((b, HEADS * P, 2 * DIM_HEAD), jnp.float32),
            jax.ShapeDtypeStruct((b, DIM, P), jnp.float32),
            jax.ShapeDtypeStruct((b, HEADS, TOPK, TOPK), jnp.int32),
        ],
    )(context, query_source, g_ctx, b_ctx, g_qs, b_qs, W_q, W_kv)

    kvflat = kvt.reshape(B * P, 2 * DIM_HEAD)
    idx2 = idx.reshape(B, 2, 128)

    kvg = _sc_gather(kvflat, idx2)
    kvg4 = kvg.reshape(b, HEADS, NSEL, 2 * DIM_HEAD)

    out = pl.pallas_call(
        _stage2_body,
        grid=(b,),
        in_specs=[
            batch3((1, INNER, P)),
            batch4((1, HEADS, NSEL, 2 * DIM_HEAD)),
            full((DIM, INNER)),
            batch3((1, DIM, P)),
            full((1, 1)),
        ],
        out_specs=batch3((1, DIM, P)),
        out_shape=jax.ShapeDtypeStruct((b, DIM, P), jnp.float32),
    )(qn, kvg4, W_out, qsn, gamma.reshape(1, 1))

    return out.reshape(b, DIM, HW, HW)


# speedup vs baseline: 1.3368x; 1.3368x over previous
import functools

import jax
import jax.numpy as jnp
from jax import lax
from jax.experimental import pallas as pl
from jax.experimental.pallas import tpu as pltpu
from jax.experimental.pallas import tpu_sc as plsc

HEADS = 8
DIM_HEAD = 64
DIM = 384
INNER = HEADS * DIM_HEAD
TOPK = 16
HW = 32
P = HW * HW
NSEL = TOPK * TOPK


def _dot(a, b, dims):
    return lax.dot_general(a, b, (dims, ((), ())),
                           preferred_element_type=jnp.float32)


def _stage1_body(ctx_ref, qs_ref, g_ctx_ref, b_ctx_ref, g_qs_ref, b_qs_ref,
                 wq_ref, wkv_ref, qn_ref, kvt_ref, qsn_ref, idx_ref):
    ctx = ctx_ref[0]
    qs = qs_ref[0]

    def chan_ln(x, g, b):
        mean = jnp.mean(x, axis=0, keepdims=True)
        var = jnp.mean((x - mean) ** 2, axis=0, keepdims=True)
        return (x - mean) * lax.rsqrt(var + 1e-5) * g + b

    ctxn = chan_ln(ctx, g_ctx_ref[...], b_ctx_ref[...])
    qsn = chan_ln(qs, g_qs_ref[...], b_qs_ref[...])
    qsn_ref[0] = qsn

    kv = _dot(wkv_ref[...], ctxn, ((1,), (0,)))
    q = _dot(wq_ref[...], qsn, ((1,), (0,)))
    k = kv[:INNER]
    v = kv[INNER:]


    i0 = jax.lax.broadcasted_iota(jnp.int32, (P, HW), 0)
    i1 = jax.lax.broadcasted_iota(jnp.int32, (P, HW), 1)
    Rm = (i0 // HW == i1).astype(jnp.float32)
    Cm = (i0 % HW == i1).astype(jnp.float32)

    hh = jax.lax.broadcasted_iota(jnp.int32, (HW, HW), 0)
    ww = jax.lax.broadcasted_iota(jnp.int32, (HW, HW), 1)
    LT = (ww < hh).astype(jnp.float32)
    Id64 = (jax.lax.broadcasted_iota(jnp.int32, (DIM_HEAD, DIM_HEAD), 0) ==
            jax.lax.broadcasted_iota(jnp.int32, (DIM_HEAD, DIM_HEAD), 1)
            ).astype(jnp.float32)
    Id16 = (jax.lax.broadcasted_iota(jnp.int32, (TOPK, TOPK), 0) ==
            jax.lax.broadcasted_iota(jnp.int32, (TOPK, TOPK), 1)
            ).astype(jnp.float32)
    ii_col = jax.lax.broadcasted_iota(jnp.int32, (HW, 1), 0).astype(jnp.float32)
    ss_row = jax.lax.broadcasted_iota(jnp.int32, (HW, TOPK), 1).astype(jnp.float32)

    pid = pl.program_id(0)
    qn_all = []
    idx_all = []
    for h in range(HEADS):
        qh = q[h * DIM_HEAD:(h + 1) * DIM_HEAD]
        kh = k[h * DIM_HEAD:(h + 1) * DIM_HEAD]
        vh = v[h * DIM_HEAD:(h + 1) * DIM_HEAD]
        qh = qh * lax.rsqrt(jnp.maximum(
            jnp.sum(qh * qh, axis=0, keepdims=True), 1e-24))
        kh = kh * lax.rsqrt(jnp.maximum(
            jnp.sum(kh * kh, axis=0, keepdims=True), 1e-24))
        qn_all.append(qh)

        kt = jnp.transpose(kh)
        vt = jnp.transpose(vh)
        kvt_ref[0, h * P:(h + 1) * P, :] = jnp.concatenate([kt, vt], axis=1)

        q_probe = jnp.sum(jnp.abs(qh), axis=1, keepdims=True)
        s_pos = jnp.sum(jnp.abs(kh) * q_probe, axis=0, keepdims=True)

        def topk_idx(sel_mat):
            s_row = _dot(s_pos, sel_mat, ((1,), (0,)))
            s_col = _dot(sel_mat, s_pos, ((0,), (1,)))
            vi = jnp.broadcast_to(s_col, (HW, HW))
            vj = jnp.broadcast_to(s_row, (HW, HW))
            beats = (vj > vi) | ((vj == vi) & (ww < hh))
            rank = jnp.sum(beats.astype(jnp.float32), axis=1, keepdims=True)
            sel = (rank < TOPK).astype(jnp.float32)
            pos = _dot(LT, sel, ((1,), (0,)))
            G = sel * (jnp.broadcast_to(pos, (HW, TOPK)) == ss_row)
            return _dot(G, ii_col, ((0,), (0,)))

        ridx = topk_idx(Rm)
        cidx = topk_idx(Cm)
        cidx_row = _dot(cidx, Id16, ((0,), (0,)))
        idx_f = ridx * float(HW) + jnp.broadcast_to(cidx_row, (TOPK, TOPK))
        base = (pid * HEADS + h) * P
        idx_all.append(idx_f.astype(jnp.int32) + base)

    qn_ref[0] = jnp.concatenate(qn_all, axis=0)
    idx_ref[0] = jnp.stack(idx_all, axis=0)


def _sc_gather(kvflat, idx):
    mesh = plsc.VectorSubcoreMesh(core_axis_name="c", subcore_axis_name="s")
    info = plsc.get_sparse_core_info()
    nc = info.num_cores

    @functools.partial(
        pl.kernel, mesh=mesh,
        out_type=jax.ShapeDtypeStruct((32, NSEL, 2 * DIM_HEAD), jnp.float32),
        scratch_types=[
            pltpu.VMEM((2, 128), jnp.int32),
            pltpu.VMEM((NSEL, 2 * DIM_HEAD), jnp.float32),
            pltpu.SemaphoreType.DMA,
        ],
    )
    def gather(kv_hbm, idx_hbm, kvg_hbm, idx_v, buf, sem):
        wid = lax.axis_index("s") * nc + lax.axis_index("c")
        pltpu.sync_copy(idx_hbm.at[wid], idx_v)
        copies = []
        for half in range(2):
            dst = pl.ds(half * 128, 128)
            copies.append(pltpu.async_copy(
                kv_hbm.at[idx_v.at[half]], buf.at[dst], sem))
        for c in copies:
            c.wait()
        pltpu.sync_copy(buf, kvg_hbm.at[wid])

    return gather(kvflat, idx)


def _stage2_body(qn_ref, kvg_ref, wout_ref, qsn_ref, gamma_ref, out_ref):
    ones_ns = jnp.ones((1, NSEL), jnp.float32)
    inner_parts = []
    for h in range(HEADS):
        qh = qn_ref[0][h * DIM_HEAD:(h + 1) * DIM_HEAD]
        kvg = kvg_ref[0, h]
        kg = kvg[:, :DIM_HEAD]
        vg = kvg[:, DIM_HEAD:]
        sim = _dot(qh, kg, ((0,), (1,)))
        e = jnp.exp(sim)
        s_row = _dot(ones_ns, e, ((1,), (1,)))
        o = _dot(vg, e, ((0,), (1,)))
        inner_parts.append(o * (1.0 / s_row))
    inner = jnp.concatenate(inner_parts, axis=0)
    proj = _dot(wout_ref[...], inner, ((1,), (0,)))
    out_ref[0] = gamma_ref[0, 0] * proj + qsn_ref[0]


def kernel(context, query_source, g_ctx, b_ctx, g_qs, b_qs, W_q, W_kv, W_out,
           gamma):
    b = context.shape[0]
    B = b * HEADS
    ctx = context.reshape(b, DIM, P)
    qs = query_source.reshape(b, DIM, P)
    g_ctx = g_ctx.reshape(DIM, 1)
    b_ctx = b_ctx.reshape(DIM, 1)
    g_qs = g_qs.reshape(DIM, 1)
    b_qs = b_qs.reshape(DIM, 1)

    full = lambda shape: pl.BlockSpec(shape, lambda i: (0,) * len(shape))
    batch3 = lambda shape: pl.BlockSpec(shape, lambda i: (i, 0, 0))
    batch4 = lambda shape: pl.BlockSpec(shape, lambda i: (i, 0, 0, 0))

    qn, kvt, qsn, idx = pl.pallas_call(
        _stage1_body,
        grid=(b,),
        in_specs=[
            batch3((1, DIM, P)), batch3((1, DIM, P)),
            full((DIM, 1)), full((DIM, 1)), full((DIM, 1)), full((DIM, 1)),
            full((INNER, DIM)), full((2 * INNER, DIM)),
        ],
        out_specs=[
            batch3((1, INNER, P)),
            batch3((1, HEADS * P, 2 * DIM_HEAD)),
            batch3((1, DIM, P)),
            batch4((1, HEADS, TOPK, TOPK)),
        ],
        out_shape=[
            jax.ShapeDtypeStruct((b, INNER, P), jnp.float32),
            jax.ShapeDtypeStruct((b, HEADS * P, 2 * DIM_HEAD), jnp.float32),
            jax.ShapeDtypeStruct((b, DIM, P), jnp.float32),
            jax.ShapeDtypeStruct((b, HEADS, TOPK, TOPK), jnp.int32),
        ],
    )(ctx, qs, g_ctx, b_ctx, g_qs, b_qs, W_q, W_kv)

    kvflat = kvt.reshape(B * P, 2 * DIM_HEAD)
    idx2 = idx.reshape(B, 2, 128)

    kvg = _sc_gather(kvflat, idx2)
    kvg4 = kvg.reshape(b, HEADS, NSEL, 2 * DIM_HEAD)

    out = pl.pallas_call(
        _stage2_body,
        grid=(b,),
        in_specs=[
            batch3((1, INNER, P)),
            batch4((1, HEADS, NSEL, 2 * DIM_HEAD)),
            full((DIM, INNER)),
            batch3((1, DIM, P)),
            full((1, 1)),
        ],
        out_specs=batch3((1, DIM, P)),
        out_shape=jax.ShapeDtypeStruct((b, DIM, P), jnp.float32),
    )(qn, kvg4, W_out, qsn, gamma.reshape(1, 1))

    return out.reshape(b, DIM, HW, HW)
